# baseline (speedup 1.0000x reference)
import functools

import jax
import jax.numpy as jnp
from jax import lax
from jax.experimental import pallas as pl
from jax.experimental.pallas import tpu as pltpu
from jax.experimental.pallas import tpu_sc as plsc

_B = 4096
_ROWW = 20
_NC = 2
_NS = 16
_L = 16
_NW = _NC * _NS
_RPW = _B // _NW
_CHUNKS = _RPW // _L
_OPW = _RPW * 4


def _u(x):
    return plsc.bitcast(x, jnp.uint32)


def _i(x):
    return plsc.bitcast(x, jnp.int32)


def _narrow_f64_bits(lo_i, hi_i):
    lo, hi = _u(lo_i), _u(hi_i)
    sign = hi & jnp.uint32(0x80000000)
    exp64 = (hi >> 20) & jnp.uint32(0x7FF)
    mant23 = ((hi & jnp.uint32(0xFFFFF)) << 3) | (lo >> 29)
    mant23 = mant23 + ((lo >> 28) & jnp.uint32(1))
    bits = sign | (((exp64 - 896) << 23) + mant23)
    bits = jnp.where(exp64 >= 897, bits, jnp.uint32(0))
    return plsc.bitcast(bits, jnp.float32)


def _widen_f32_bits(v):
    m = _u(v)
    sign = m & jnp.uint32(0x80000000)
    e = (m >> 23) & jnp.uint32(0xFF)
    mant = m & jnp.uint32(0x7FFFFF)
    hi = sign | ((e + 896) << 20) | (mant >> 3)
    lo = mant << 29
    return _i(lo), _i(hi)


def _sc_body(phi_hbm, w_hbm, out_hbm, pv, wv, ov):
    wid = lax.axis_index("s") * _NC + lax.axis_index("c")
    pltpu.sync_copy(phi_hbm.at[pl.ds(wid * (_RPW * _ROWW), _RPW * _ROWW)], pv)
    pltpu.sync_copy(w_hbm, wv)
    w_vecs = []
    for c in range(6):
        lo = plsc.load_gather(wv, [jnp.full((_L,), 2 * c, jnp.int32)])
        hi = plsc.load_gather(wv, [jnp.full((_L,), 2 * c + 1, jnp.int32)])
        w_vecs.append(_narrow_f64_bits(lo, hi))
    row_off = lax.iota(jnp.int32, _L) * _ROWW
    quad_off = lax.iota(jnp.int32, _L) * 4
    for chunk in range(_CHUNKS):
        base = chunk * _L * _ROWW
        d = jnp.zeros((_L,), jnp.float32)
        for c in range(6):
            left = plsc.load_gather(pv, [row_off + (base + c)])
            right = plsc.load_gather(pv, [row_off + (base + 10 + c)])
            d = d + (left - right) * w_vecs[c]
        s = 1.0 / (1.0 + jnp.exp(-d))
        s_lo, s_hi = _widen_f32_bits(s)
        r_lo, r_hi = _widen_f32_bits(1.0 - s)
        oidx = quad_off + chunk * _L * 4
        plsc.store_scatter(ov, [oidx], s_lo)
        plsc.store_scatter(ov, [oidx + 1], s_hi)
        plsc.store_scatter(ov, [oidx + 2], r_lo)
        plsc.store_scatter(ov, [oidx + 3], r_hi)
    pltpu.sync_copy(ov, out_hbm.at[pl.ds(wid * _OPW, _OPW)])


@functools.lru_cache(maxsize=1)
def _sc_call():
    mesh = plsc.VectorSubcoreMesh(core_axis_name="c", subcore_axis_name="s")
    return pl.kernel(
        _sc_body,
        mesh=mesh,
        compiler_params=pltpu.CompilerParams(needs_layout_passes=False),
        out_type=jax.ShapeDtypeStruct((_B * 4,), jnp.int32),
        scratch_types=[
            pltpu.VMEM((_RPW * _ROWW,), jnp.float32),
            pltpu.VMEM((12,), jnp.int32),
            pltpu.VMEM((_OPW,), jnp.int32),
        ],
    )


def kernel(phi, succ_feats, W):
    del succ_feats
    phi32 = phi.reshape(_B * _ROWW)
    w_bits = lax.bitcast_convert_type(W, jnp.int32).reshape(12)
    out = _sc_call()(phi32, w_bits)
    return lax.bitcast_convert_type(out.reshape(_B, 2, 1, 2), jnp.float64)


# speedup vs baseline: 1.0036x; 1.0036x over previous
import functools

import jax
import jax.numpy as jnp
from jax import lax
from jax.experimental import pallas as pl
from jax.experimental.pallas import tpu as pltpu
from jax.experimental.pallas import tpu_sc as plsc

_B = 4096
_ROWW = 20
_NC = 2
_NS = 16
_L = 16
_NW = _NC * _NS
_RPW = _B // _NW
_CHUNKS = _RPW // _L
_OPW = _RPW * 4


def _u(x):
    return plsc.bitcast(x, jnp.uint32)


def _i(x):
    return plsc.bitcast(x, jnp.int32)


def _widen_f32_bits(v):
    m = _u(v)
    sign = m & jnp.uint32(0x80000000)
    e = (m >> 23) & jnp.uint32(0xFF)
    mant = m & jnp.uint32(0x7FFFFF)
    hi = sign | ((e + 896) << 20) | (mant >> 3)
    lo = mant << 29
    return _i(lo), _i(hi)


def _sc_body(phi_hbm, w_hbm, out_hbm, pv, wv, ov):
    wid = lax.axis_index("s") * _NC + lax.axis_index("c")
    pltpu.sync_copy(phi_hbm.at[pl.ds(wid * (_RPW * _ROWW), _RPW * _ROWW)], pv)
    pltpu.sync_copy(w_hbm, wv)
    row_off = lax.iota(jnp.int32, _L) * _ROWW
    quad_off = lax.iota(jnp.int32, _L) * 4
    for chunk in range(_CHUNKS):
        base = chunk * _L * _ROWW
        d = jnp.zeros((_L,), jnp.float32)
        for c in range(6):
            left = plsc.load_gather(pv, [row_off + (base + c)])
            right = plsc.load_gather(pv, [row_off + (base + 10 + c)])
            d = d + (left - right) * wv[c]
        s = 1.0 / (1.0 + jnp.exp(-d))
        s_lo, s_hi = _widen_f32_bits(s)
        r_lo, r_hi = _widen_f32_bits(1.0 - s)
        oidx = quad_off + chunk * _L * 4
        plsc.store_scatter(ov, [oidx], s_lo)
        plsc.store_scatter(ov, [oidx + 1], s_hi)
        plsc.store_scatter(ov, [oidx + 2], r_lo)
        plsc.store_scatter(ov, [oidx + 3], r_hi)
    pltpu.sync_copy(ov, out_hbm.at[pl.ds(wid * _OPW, _OPW)])


@functools.lru_cache(maxsize=1)
def _sc_call():
    mesh = plsc.VectorSubcoreMesh(core_axis_name="c", subcore_axis_name="s")
    return pl.kernel(
        _sc_body,
        mesh=mesh,
        compiler_params=pltpu.CompilerParams(needs_layout_passes=False),
        out_type=jax.ShapeDtypeStruct((_B * 4,), jnp.int32),
        scratch_types=[
            pltpu.VMEM((_RPW * _ROWW,), jnp.float32),
            pltpu.VMEM((6, _L), jnp.float32),
            pltpu.VMEM((_OPW,), jnp.int32),
        ],
    )


def kernel(phi, succ_feats, W):
    del succ_feats
    phi32 = phi.reshape(_B * _ROWW)
    wsp = jnp.broadcast_to(W.astype(jnp.float32).reshape(6, 1), (6, _L))
    out = _sc_call()(phi32, wsp)
    return lax.bitcast_convert_type(out.reshape(_B, 2, 1, 2), jnp.float64)


# speedup vs baseline: 4.6941x; 4.6771x over previous
import functools

import jax
import jax.numpy as jnp
from jax import lax
from jax.experimental import pallas as pl
from jax.experimental.pallas import tpu as pltpu
from jax.experimental.pallas import tpu_sc as plsc

_B = 4096
_ROWW = 20
_NC = 2
_NS = 16
_L = 16
_NW = _NC * _NS
_RPW = _B // _NW
_CHUNKS = _RPW // _L


def _sc_body(phi_hbm, w_hbm, out_hbm, pv, wv, ov):
    wid = lax.axis_index("s") * _NC + lax.axis_index("c")
    pltpu.sync_copy(phi_hbm.at[pl.ds(wid * (_RPW * _ROWW), _RPW * _ROWW)], pv)
    pltpu.sync_copy(w_hbm, wv)
    row_off = lax.iota(jnp.int32, _L) * _ROWW
    pair_off = lax.iota(jnp.int32, _L) * 2
    for chunk in range(_CHUNKS):
        base = chunk * _L * _ROWW
        d = jnp.zeros((_L,), jnp.float32)
        for c in range(6):
            left = plsc.load_gather(pv, [row_off + (base + c)])
            right = plsc.load_gather(pv, [row_off + (base + 10 + c)])
            d = d + (left - right) * wv[c]
        s = 1.0 / (1.0 + jnp.exp(-d))
        oidx = pair_off + chunk * _L * 2
        plsc.store_scatter(ov, [oidx], s)
        plsc.store_scatter(ov, [oidx + 1], 1.0 - s)
    pltpu.sync_copy(ov, out_hbm.at[pl.ds(wid * (_RPW * 2), _RPW * 2)])


@functools.lru_cache(maxsize=1)
def _sc_call():
    mesh = plsc.VectorSubcoreMesh(core_axis_name="c", subcore_axis_name="s")
    return pl.kernel(
        _sc_body,
        mesh=mesh,
        compiler_params=pltpu.CompilerParams(needs_layout_passes=False),
        out_type=jax.ShapeDtypeStruct((_B * 2,), jnp.float32),
        scratch_types=[
            pltpu.VMEM((_RPW * _ROWW,), jnp.float32),
            pltpu.VMEM((6, _L), jnp.float32),
            pltpu.VMEM((_RPW * 2,), jnp.float32),
        ],
    )


def kernel(phi, succ_feats, W):
    del succ_feats
    phi32 = phi.astype(jnp.float32).reshape(_B * _ROWW)
    wsp = jnp.broadcast_to(W.astype(jnp.float32).reshape(6, 1), (6, _L))
    out = _sc_call()(phi32, wsp)
    return out.reshape(_B, 2, 1).astype(jnp.float64)


# speedup vs baseline: 4.7844x; 1.0192x over previous
import functools

import jax
import jax.numpy as jnp
from jax import lax
from jax.experimental import pallas as pl
from jax.experimental.pallas import tpu as pltpu
from jax.experimental.pallas import tpu_sc as plsc

_B = 4096
_ROWW = 12
_NC = 2
_NS = 16
_L = 16
_NW = _NC * _NS
_RPW = _B // _NW
_CHUNKS = _RPW // _L


def _sc_body(phi_hbm, w_hbm, out_hbm, pv, wv, ov):
    wid = lax.axis_index("s") * _NC + lax.axis_index("c")
    pltpu.sync_copy(phi_hbm.at[pl.ds(wid * (_RPW * _ROWW), _RPW * _ROWW)], pv)
    pltpu.sync_copy(w_hbm, wv)
    row_off = lax.iota(jnp.int32, _L) * _ROWW
    pair_off = lax.iota(jnp.int32, _L) * 2
    for chunk in range(_CHUNKS):
        base = chunk * _L * _ROWW
        d = jnp.zeros((_L,), jnp.float32)
        for c in range(6):
            left = plsc.load_gather(pv, [row_off + (base + c)])
            right = plsc.load_gather(pv, [row_off + (base + 6 + c)])
            d = d + (left - right) * wv[c]
        s = 1.0 / (1.0 + jnp.exp(-d))
        oidx = pair_off + chunk * _L * 2
        plsc.store_scatter(ov, [oidx], s)
        plsc.store_scatter(ov, [oidx + 1], 1.0 - s)
    pltpu.sync_copy(ov, out_hbm.at[pl.ds(wid * (_RPW * 2), _RPW * 2)])


@functools.lru_cache(maxsize=1)
def _sc_call():
    mesh = plsc.VectorSubcoreMesh(core_axis_name="c", subcore_axis_name="s")
    return pl.kernel(
        _sc_body,
        mesh=mesh,
        compiler_params=pltpu.CompilerParams(needs_layout_passes=False),
        out_type=jax.ShapeDtypeStruct((_B * 2,), jnp.float32),
        scratch_types=[
            pltpu.VMEM((_RPW * _ROWW,), jnp.float32),
            pltpu.VMEM((6, _L), jnp.float32),
            pltpu.VMEM((_RPW * 2,), jnp.float32),
        ],
    )


def kernel(phi, succ_feats, W):
    del succ_feats
    phi12 = phi[:, :, 0:6].astype(jnp.float32).reshape(_B * _ROWW)
    wsp = jnp.broadcast_to(W.astype(jnp.float32).reshape(6, 1), (6, _L))
    out = _sc_call()(phi12, wsp)
    return out.astype(jnp.float64).reshape(_B, 2, 1)


# speedup vs baseline: 5.2607x; 1.0996x over previous
import functools

import jax
import jax.numpy as jnp
from jax import lax
from jax.experimental import pallas as pl
from jax.experimental.pallas import tpu as pltpu
from jax.experimental.pallas import tpu_sc as plsc

_B = 4096
_T = 2
_C = 10
_NC = 2
_NS = 16
_L = 16
_NW = _NC * _NS
_RPW = _B // _NW
_CHUNKS = _RPW // _L


def _sc_body(phi_hbm, w_hbm, out_hbm, pv, wv, ov):
    wid = lax.axis_index("s") * _NC + lax.axis_index("c")
    pltpu.sync_copy(phi_hbm.at[pl.ds(wid * _RPW, _RPW), :, :], pv)
    pltpu.sync_copy(w_hbm, wv)
    lane = lax.iota(jnp.int32, _L)
    pair_off = lane * 2
    for chunk in range(_CHUNKS):
        rows = lane + chunk * _L
        d = jnp.zeros((_L,), jnp.float32)
        for c in range(6):
            cv = jnp.full((_L,), c, jnp.int32)
            left = plsc.load_gather(pv, [rows, jnp.zeros((_L,), jnp.int32), cv])
            right = plsc.load_gather(pv, [rows, jnp.ones((_L,), jnp.int32), cv])
            d = d + (left - right) * wv[c]
        s = 1.0 / (1.0 + jnp.exp(-d))
        oidx = pair_off + chunk * _L * 2
        plsc.store_scatter(ov, [oidx], s)
        plsc.store_scatter(ov, [oidx + 1], 1.0 - s)
    pltpu.sync_copy(ov, out_hbm.at[pl.ds(wid * (_RPW * 2), _RPW * 2)])


@functools.lru_cache(maxsize=1)
def _sc_call():
    mesh = plsc.VectorSubcoreMesh(core_axis_name="c", subcore_axis_name="s")
    return pl.kernel(
        _sc_body,
        mesh=mesh,
        compiler_params=pltpu.CompilerParams(needs_layout_passes=False),
        out_type=jax.ShapeDtypeStruct((_B * 2,), jnp.float32),
        scratch_types=[
            pltpu.VMEM((_RPW, _T, _C), jnp.float32),
            pltpu.VMEM((6, _L), jnp.float32),
            pltpu.VMEM((_RPW * 2,), jnp.float32),
        ],
    )


def kernel(phi, succ_feats, W):
    del succ_feats
    wsp = jnp.broadcast_to(W.astype(jnp.float32).reshape(6, 1), (6, _L))
    out = _sc_call()(phi, wsp)
    return out.astype(jnp.float64).reshape(_B, 2, 1)
